# Initial kernel scaffold; baseline (speedup 1.0000x reference)
import functools

import jax
import jax.numpy as jnp
from jax import lax
from jax.experimental import pallas as pl
from jax.experimental.pallas import tpu as pltpu, tpu_sc as plsc

B, C, T, NV, KT = 4, 32, 12, 10000, 3
T1 = T - (KT - 1)
T2 = T1 - (KT - 1)
F = B * C * T1
FC = 256
NFC = F // FC
VPT = 320
NW = 32
NVP = NW * VPT
ECH = 512
EB = 16
EPAD = 1024


def _spmm_body(z_hbm, rows_hbm, cols_hbm, vals_hbm, prm_hbm, o_hbm,
               prm_v, rows_v, cols_v, vals_v, buf0, buf1, acc_v, sem0, sem1):
    wid = lax.axis_index("s") * 2 + lax.axis_index("c")
    pltpu.sync_copy(prm_hbm, prm_v)
    estart = prm_v[pl.ds(wid, 16)][0]
    eend = prm_v[pl.ds(wid + 32, 16)][0]
    vs = wid * VPT
    e0 = (estart // 8) * 8
    ne = eend - e0
    nch = (ne + ECH - 1) // ECH

    def process(off, eb, buf):
        r16 = jnp.clip(rows_v[pl.ds(off, EB)] - vs, 0, VPT - 1)
        eidx = lax.iota(jnp.int32, EB) + (eb + off)
        valid = (eidx >= estart) & (eidx < eend)
        v16 = jnp.where(valid, vals_v[pl.ds(off, EB)], 0.0)
        for i in range(EB):
            r = r16[i]
            val = v16[i]
            for j in range(FC // 16):
                sl = pl.ds(j * 16, 16)
                plsc.addupdate(acc_v.at[r, sl], val * buf[i, sl])

    def fc_body(fc, _):
        fco = pl.multiple_of(fc * FC, FC)

        def zr(r, _):
            for j in range(FC // 16):
                acc_v[r, pl.ds(j * 16, 16)] = jnp.zeros((16,), jnp.float32)
            return 0
        lax.fori_loop(0, VPT, zr, 0)

        def ch_body(ch, _):
            eb = e0 + ch * ECH
            pltpu.sync_copy(rows_hbm.at[pl.ds(eb, ECH)], rows_v)
            pltpu.sync_copy(cols_hbm.at[pl.ds(eb, ECH + EB)], cols_v)
            pltpu.sync_copy(vals_hbm.at[pl.ds(eb, ECH)], vals_v)
            pltpu.async_copy(
                z_hbm.at[cols_v.at[pl.ds(0, EB)], pl.ds(fco, FC)], buf0, sem0)

            def pair(p, _):
                o0 = p * 2 * EB
                pltpu.async_copy(
                    z_hbm.at[cols_v.at[pl.ds(o0 + EB, EB)], pl.ds(fco, FC)],
                    buf1, sem1)
                pltpu.make_async_copy(
                    z_hbm.at[cols_v.at[pl.ds(0, EB)], pl.ds(fco, FC)],
                    buf0, sem0).wait()
                process(o0, eb, buf0)
                pltpu.async_copy(
                    z_hbm.at[cols_v.at[pl.ds(o0 + 2 * EB, EB)], pl.ds(fco, FC)],
                    buf0, sem0)
                pltpu.make_async_copy(
                    z_hbm.at[cols_v.at[pl.ds(0, EB)], pl.ds(fco, FC)],
                    buf1, sem1).wait()
                process(o0 + EB, eb, buf1)
                return 0
            lax.fori_loop(0, ECH // (2 * EB), pair, 0)
            pltpu.make_async_copy(
                z_hbm.at[cols_v.at[pl.ds(0, EB)], pl.ds(fco, FC)],
                buf0, sem0).wait()
            return 0
        lax.fori_loop(0, nch, ch_body, 0)
        pltpu.sync_copy(acc_v, o_hbm.at[pl.ds(vs, VPT), pl.ds(fco, FC)])
        return 0
    lax.fori_loop(0, NFC, fc_body, 0)


def _spmm(z, rows_p, cols_p, vals_p, params):
    mesh = plsc.VectorSubcoreMesh(core_axis_name="c", subcore_axis_name="s")
    return pl.kernel(
        _spmm_body, mesh=mesh,
        out_type=jax.ShapeDtypeStruct((NVP, F), jnp.float32),
        scratch_types=[
            pltpu.VMEM((80,), jnp.int32),
            pltpu.VMEM((ECH,), jnp.int32),
            pltpu.VMEM((ECH + EB,), jnp.int32),
            pltpu.VMEM((ECH,), jnp.float32),
            pltpu.VMEM((EB, FC), jnp.float32),
            pltpu.VMEM((EB, FC), jnp.float32),
            pltpu.VMEM((VPT, FC), jnp.float32),
            pltpu.SemaphoreType.DMA,
            pltpu.SemaphoreType.DMA,
        ],
    )(z, rows_p, cols_p, vals_p, params)


def _conv1_glu_body(x0_ref, x1_ref, x2_ref, w0_ref, w1_ref, w2_ref, b_ref,
                    o_ref):
    x0 = x0_ref[0, :, 0, :]
    x1 = x1_ref[0, :, 0, :]
    x2 = x2_ref[0, :, 0, :]
    xc = (jnp.dot(w0_ref[...], x0, preferred_element_type=jnp.float32)
          + jnp.dot(w1_ref[...], x1, preferred_element_type=jnp.float32)
          + jnp.dot(w2_ref[...], x2, preferred_element_type=jnp.float32)
          + b_ref[...])
    p = xc[:C, :]
    q = xc[C:, :]
    o_ref[0, :, 0, :] = (p + x2) * jax.nn.sigmoid(q)


def _conv1_glu(x, conv1_w, conv1_b):
    w = [conv1_w[:, :, k, 0] for k in range(KT)]
    bias = conv1_b[:, None]
    xspec = lambda k: pl.BlockSpec((1, C, 1, NV), lambda b, t, k=k: (b, 0, t + k, 0))
    wspec = pl.BlockSpec((2 * C, C), lambda b, t: (0, 0))
    return pl.pallas_call(
        _conv1_glu_body,
        grid=(B, T1),
        in_specs=[xspec(0), xspec(1), xspec(2), wspec, wspec, wspec,
                  pl.BlockSpec((2 * C, 1), lambda b, t: (0, 0))],
        out_specs=pl.BlockSpec((1, C, 1, NV), lambda b, t: (b, 0, t, 0)),
        out_shape=jax.ShapeDtypeStruct((B, C, T1, NV), jnp.float32),
    )(x, x, x, w[0], w[1], w[2], bias)


def _tail_body(o0_ref, o1_ref, o2_ref, y0_ref, y1_ref, y2_ref, gw_ref,
               gb_ref, w20_ref, w21_ref, w22_ref, b2_ref, gma_ref, bta_ref,
               out_ref):
    gw = gw_ref[...]
    gb = gb_ref[...]

    def xr(o_ref, y_ref):
        g = jnp.dot(o_ref[0, 0], gw, preferred_element_type=jnp.float32) + gb
        return jnp.maximum(g + y_ref[0, 0], 0.0)

    xr0 = xr(o0_ref, y0_ref)
    xr1 = xr(o1_ref, y1_ref)
    xr2 = xr(o2_ref, y2_ref)
    y = (jnp.dot(xr0, w20_ref[...], preferred_element_type=jnp.float32)
         + jnp.dot(xr1, w21_ref[...], preferred_element_type=jnp.float32)
         + jnp.dot(xr2, w22_ref[...], preferred_element_type=jnp.float32)
         + b2_ref[...])
    y = jnp.maximum(y + xr2, 0.0)
    mean = jnp.mean(y)
    var = jnp.mean((y - mean) ** 2)
    out_ref[0, 0] = ((y - mean) * lax.rsqrt(var + 1e-5) * gma_ref[...]
                     + bta_ref[...])


def _tail(o4, x1_fl, gcn_w, gcn_b, conv2_w, conv2_b, ln_gamma, ln_beta):
    w2 = [conv2_w[:, :, k, 0].T for k in range(KT)]
    ospec = lambda k: pl.BlockSpec((1, 1, NV, C), lambda b, t, k=k: (b, t + k, 0, 0))
    cspec = pl.BlockSpec((C, C), lambda b, t: (0, 0))
    rspec = pl.BlockSpec((1, C), lambda b, t: (0, 0))
    gspec = pl.BlockSpec((NV, C), lambda b, t: (0, 0))
    return pl.pallas_call(
        _tail_body,
        grid=(B, T2),
        in_specs=[ospec(0), ospec(1), ospec(2), ospec(0), ospec(1), ospec(2),
                  cspec, rspec, cspec, cspec, cspec, rspec, gspec, gspec],
        out_specs=pl.BlockSpec((1, 1, NV, C), lambda b, t: (b, t, 0, 0)),
        out_shape=jax.ShapeDtypeStruct((B, T2, NV, C), jnp.float32),
    )(o4, o4, o4, x1_fl, x1_fl, x1_fl, gcn_w, gcn_b[None, :], w2[0], w2[1],
      w2[2], conv2_b[None, :], ln_gamma, ln_beta)


def kernel(x, conv1_w, conv1_b, gcn_w, gcn_b, conv2_w, conv2_b,
           ln_gamma, ln_beta, filter_vals, filter_rows, filter_cols):
    order = jnp.argsort(filter_rows)
    rows_s = filter_rows[order]
    cols_s = filter_cols[order]
    vals_s = filter_vals[order]
    bounds = jnp.arange(NW + 1, dtype=jnp.int32) * VPT
    ptr = jnp.searchsorted(rows_s, bounds, side="left").astype(jnp.int32)
    params = jnp.zeros((80,), jnp.int32)
    params = params.at[0:NW].set(ptr[:NW]).at[NW:2 * NW].set(ptr[1:NW + 1])
    rows_p = jnp.pad(rows_s, (0, EPAD))
    cols_p = jnp.pad(cols_s, (0, EPAD))
    vals_p = jnp.pad(vals_s, (0, EPAD))

    x1 = _conv1_glu(x, conv1_w, conv1_b)
    z = x1.reshape(NV, F)
    o = _spmm(z, rows_p, cols_p, vals_p, params)
    o4 = o[:NV].reshape(B, T1, NV, C)
    x1_fl = x1.transpose(0, 2, 3, 1)
    out_fl = _tail(o4, x1_fl, gcn_w, gcn_b, conv2_w, conv2_b,
                   ln_gamma, ln_beta)
    return out_fl.transpose(0, 3, 1, 2)


# speedup vs baseline: 1.1225x; 1.1225x over previous
import functools

import jax
import jax.numpy as jnp
from jax import lax
from jax.experimental import pallas as pl
from jax.experimental.pallas import tpu as pltpu, tpu_sc as plsc

B, C, T, NV, KT = 4, 32, 12, 10000, 3
T1 = T - (KT - 1)
T2 = T1 - (KT - 1)
F = B * C * T1
FC = 256
NFC = F // FC
VPT = 320
NW = 32
NVP = NW * VPT
ECH = 512
EB = 16
EPAD = 1024


def _spmm_body(z_hbm, rows_hbm, cols_hbm, vals_hbm, prm_hbm, o_hbm,
               prm_v, rows_v, cols_v, vals_v, buf0, buf1, acc_v, sem0, sem1):
    wid = lax.axis_index("s") * 2 + lax.axis_index("c")
    pltpu.sync_copy(prm_hbm, prm_v)
    estart = prm_v[pl.ds(wid, 16)][0]
    eend = prm_v[pl.ds(wid + 32, 16)][0]
    vs = wid * VPT
    e0 = (estart // 8) * 8
    ne = eend - e0
    nch = (ne + ECH - 1) // ECH

    def process(off, eb, buf):
        r16 = jnp.clip(rows_v[pl.ds(off, EB)] - vs, 0, VPT - 1)
        eidx = lax.iota(jnp.int32, EB) + (eb + off)
        valid = (eidx >= estart) & (eidx < eend)
        v16 = jnp.where(valid, vals_v[pl.ds(off, EB)], 0.0)
        for i in range(EB):
            r = r16[i]
            val = v16[i]
            for j in range(FC // 16):
                sl = pl.ds(j * 16, 16)
                plsc.addupdate(acc_v.at[r, sl], val * buf[i, sl])

    def fc_body(fc, _):
        fco = pl.multiple_of(fc * FC, FC)

        def zr(r, _):
            for j in range(FC // 16):
                acc_v[r, pl.ds(j * 16, 16)] = jnp.zeros((16,), jnp.float32)
            return 0
        lax.fori_loop(0, VPT, zr, 0)

        def ch_body(ch, _):
            eb = e0 + ch * ECH
            pltpu.sync_copy(rows_hbm.at[pl.ds(eb, ECH)], rows_v)
            pltpu.sync_copy(cols_hbm.at[pl.ds(eb, ECH + EB)], cols_v)
            pltpu.sync_copy(vals_hbm.at[pl.ds(eb, ECH)], vals_v)
            pltpu.async_copy(
                z_hbm.at[cols_v.at[pl.ds(0, EB)], pl.ds(fco, FC)], buf0, sem0)

            def pair(p, _):
                o0 = p * 2 * EB
                pltpu.async_copy(
                    z_hbm.at[cols_v.at[pl.ds(o0 + EB, EB)], pl.ds(fco, FC)],
                    buf1, sem1)
                pltpu.make_async_copy(
                    z_hbm.at[cols_v.at[pl.ds(0, EB)], pl.ds(fco, FC)],
                    buf0, sem0).wait()
                process(o0, eb, buf0)
                pltpu.async_copy(
                    z_hbm.at[cols_v.at[pl.ds(o0 + 2 * EB, EB)], pl.ds(fco, FC)],
                    buf0, sem0)
                pltpu.make_async_copy(
                    z_hbm.at[cols_v.at[pl.ds(0, EB)], pl.ds(fco, FC)],
                    buf1, sem1).wait()
                process(o0 + EB, eb, buf1)
                return 0
            lax.fori_loop(0, ECH // (2 * EB), pair, 0)
            pltpu.make_async_copy(
                z_hbm.at[cols_v.at[pl.ds(0, EB)], pl.ds(fco, FC)],
                buf0, sem0).wait()
            return 0
        lax.fori_loop(0, nch, ch_body, 0)
        pltpu.sync_copy(acc_v, o_hbm.at[pl.ds(vs, VPT), pl.ds(fco, FC)])
        return 0
    lax.fori_loop(0, NFC, fc_body, 0)


def _spmm(z, rows_p, cols_p, vals_p, params):
    mesh = plsc.VectorSubcoreMesh(core_axis_name="c", subcore_axis_name="s")
    return pl.kernel(
        _spmm_body, mesh=mesh,
        out_type=jax.ShapeDtypeStruct((NVP, F), jnp.float32),
        scratch_types=[
            pltpu.VMEM((80,), jnp.int32),
            pltpu.VMEM((ECH,), jnp.int32),
            pltpu.VMEM((ECH + EB,), jnp.int32),
            pltpu.VMEM((ECH,), jnp.float32),
            pltpu.VMEM((EB, FC), jnp.float32),
            pltpu.VMEM((EB, FC), jnp.float32),
            pltpu.VMEM((VPT, FC), jnp.float32),
            pltpu.SemaphoreType.DMA,
            pltpu.SemaphoreType.DMA,
        ],
    )(z, rows_p, cols_p, vals_p, params)


def _conv1_glu_body(x_ref, w0_ref, w1_ref, w2_ref, b_ref, o_ref):
    w0 = w0_ref[...]
    w1 = w1_ref[...]
    w2 = w2_ref[...]
    bias = b_ref[...]
    for t in range(T1):
        x0 = x_ref[0, :, t, :]
        x1 = x_ref[0, :, t + 1, :]
        x2 = x_ref[0, :, t + 2, :]
        xc = (jnp.dot(w0, x0, preferred_element_type=jnp.float32)
              + jnp.dot(w1, x1, preferred_element_type=jnp.float32)
              + jnp.dot(w2, x2, preferred_element_type=jnp.float32)
              + bias)
        o_ref[0, :, t, :] = (xc[:C, :] + x2) * jax.nn.sigmoid(xc[C:, :])


_NVB = 1280


def _conv1_glu(x, conv1_w, conv1_b):
    w = [conv1_w[:, :, k, 0] for k in range(KT)]
    bias = conv1_b[:, None]
    wspec = pl.BlockSpec((2 * C, C), lambda b, v: (0, 0))
    nvb = (NV + _NVB - 1) // _NVB
    return pl.pallas_call(
        _conv1_glu_body,
        grid=(B, nvb),
        in_specs=[pl.BlockSpec((1, C, T, _NVB), lambda b, v: (b, 0, 0, v)),
                  wspec, wspec, wspec,
                  pl.BlockSpec((2 * C, 1), lambda b, v: (0, 0))],
        out_specs=pl.BlockSpec((1, C, T1, _NVB), lambda b, v: (b, 0, 0, v)),
        out_shape=jax.ShapeDtypeStruct((B, C, T1, NV), jnp.float32),
    )(x, w[0], w[1], w[2], bias)


_NVT = 2000


def _tail_body(o0_ref, o1_ref, o2_ref, y0_ref, y1_ref, y2_ref, gw_ref,
               gb_ref, w20_ref, w21_ref, w22_ref, b2_ref, out_ref):
    gw = gw_ref[...]
    gb = gb_ref[...]

    def xr(o_ref, y_ref):
        g = jnp.dot(o_ref[0, 0], gw, preferred_element_type=jnp.float32) + gb
        return jnp.maximum(g + y_ref[0, 0], 0.0)

    xr0 = xr(o0_ref, y0_ref)
    xr1 = xr(o1_ref, y1_ref)
    xr2 = xr(o2_ref, y2_ref)
    y = (jnp.dot(xr0, w20_ref[...], preferred_element_type=jnp.float32)
         + jnp.dot(xr1, w21_ref[...], preferred_element_type=jnp.float32)
         + jnp.dot(xr2, w22_ref[...], preferred_element_type=jnp.float32)
         + b2_ref[...])
    out_ref[0, 0] = jnp.maximum(y + xr2, 0.0)


def _tail(o4, x1_fl, gcn_w, gcn_b, conv2_w, conv2_b):
    w2 = [conv2_w[:, :, k, 0].T for k in range(KT)]
    ospec = lambda k: pl.BlockSpec(
        (1, 1, _NVT, C), lambda b, t, v, k=k: (b, t + k, v, 0))
    cspec = pl.BlockSpec((C, C), lambda b, t, v: (0, 0))
    rspec = pl.BlockSpec((1, C), lambda b, t, v: (0, 0))
    return pl.pallas_call(
        _tail_body,
        grid=(B, T2, NV // _NVT),
        in_specs=[ospec(0), ospec(1), ospec(2), ospec(0), ospec(1), ospec(2),
                  cspec, rspec, cspec, cspec, cspec, rspec],
        out_specs=pl.BlockSpec((1, 1, _NVT, C), lambda b, t, v: (b, t, v, 0)),
        out_shape=jax.ShapeDtypeStruct((B, T2, NV, C), jnp.float32),
    )(o4, o4, o4, x1_fl, x1_fl, x1_fl, gcn_w, gcn_b[None, :], w2[0], w2[1],
      w2[2], conv2_b[None, :])


def _ln_body(y_ref, gma_ref, bta_ref, out_ref):
    y = y_ref[0, 0]
    mean = jnp.mean(y)
    var = jnp.mean((y - mean) ** 2)
    out_ref[0, 0] = ((y - mean) * lax.rsqrt(var + 1e-5) * gma_ref[...]
                     + bta_ref[...])


def _layernorm(y, ln_gamma, ln_beta):
    gspec = pl.BlockSpec((NV, C), lambda b, t: (0, 0))
    return pl.pallas_call(
        _ln_body,
        grid=(B, T2),
        in_specs=[pl.BlockSpec((1, 1, NV, C), lambda b, t: (b, t, 0, 0)),
                  gspec, gspec],
        out_specs=pl.BlockSpec((1, 1, NV, C), lambda b, t: (b, t, 0, 0)),
        out_shape=jax.ShapeDtypeStruct((B, T2, NV, C), jnp.float32),
    )(y, ln_gamma, ln_beta)


def kernel(x, conv1_w, conv1_b, gcn_w, gcn_b, conv2_w, conv2_b,
           ln_gamma, ln_beta, filter_vals, filter_rows, filter_cols):
    order = jnp.argsort(filter_rows)
    rows_s = filter_rows[order]
    cols_s = filter_cols[order]
    vals_s = filter_vals[order]
    bounds = jnp.arange(NW + 1, dtype=jnp.int32) * VPT
    ptr = jnp.searchsorted(rows_s, bounds, side="left").astype(jnp.int32)
    params = jnp.zeros((80,), jnp.int32)
    params = params.at[0:NW].set(ptr[:NW]).at[NW:2 * NW].set(ptr[1:NW + 1])
    rows_p = jnp.pad(rows_s, (0, EPAD))
    cols_p = jnp.pad(cols_s, (0, EPAD))
    vals_p = jnp.pad(vals_s, (0, EPAD))

    x1 = _conv1_glu(x, conv1_w, conv1_b)
    z = x1.reshape(NV, F)
    o = _spmm(z, rows_p, cols_p, vals_p, params)
    o4 = o[:NV].reshape(B, T1, NV, C)
    x1_fl = x1.transpose(0, 2, 3, 1)
    y = _tail(o4, x1_fl, gcn_w, gcn_b, conv2_w, conv2_b)
    out_fl = _layernorm(y, ln_gamma, ln_beta)
    return out_fl.transpose(0, 3, 1, 2)
